# baseline (speedup 1.0000x reference)
import functools

import jax
import jax.numpy as jnp
from jax import lax
from jax.experimental import pallas as pl
from jax.experimental.pallas import tpu as pltpu
from jax.experimental.pallas import tpu_sc as plsc

_BATCH = 16384
_D = 64
_LANES = 16
_NC = 2
_NS = 16
_NW = _NC * _NS
_BPW = _BATCH // _NW
_CHUNK = 128
_NCHUNK = _BPW // _CHUNK

_NE = 1000000
_BE = 4096
_KE = 503808
_E0 = _KE + 121 * _BE
_NT = 640
_KR = 512

_BITREV = [0, 8, 4, 12, 2, 10, 6, 14, 1, 9, 5, 13, 3, 11, 7, 15]


def _eye64():
    return (lax.broadcasted_iota(jnp.int32, (_D, _D), 0)
            == lax.broadcasted_iota(jnp.int32, (_D, _D), 1)).astype(jnp.float32)


def _mxu_t(x):
    return lax.dot_general(x, _eye64(), (((0,), (0,)), ((), ())),
                           precision=lax.Precision.HIGHEST,
                           preferred_element_type=jnp.float32)


def _relayout_body(a_ref, b_ref, o_ref):
    o_ref[...] = jnp.concatenate([_mxu_t(a_ref[...]), _mxu_t(b_ref[...])], axis=1)


def _two_halves(table, k, be, nblk_in):
    t = table.T
    grid = k // be
    return pl.pallas_call(
        _relayout_body,
        grid=(grid,),
        in_specs=[pl.BlockSpec((_D, be), lambda i: (0, i)),
                  pl.BlockSpec((_D, be),
                               lambda i: (0, jnp.minimum(i + grid, nblk_in - 1)))],
        out_specs=pl.BlockSpec((be, 2 * _D), lambda i: (i, 0)),
        out_shape=jax.ShapeDtypeStruct((k, 2 * _D), jnp.float32),
    )(t, t)


def _tail_body(a_ref, o_ref):
    o_ref[...] = jnp.concatenate(
        [_mxu_t(a_ref[...]), jnp.zeros((128, _D), jnp.float32)], axis=1)


def _tail_table(table):
    t = table.T
    base = _E0 // 128
    return pl.pallas_call(
        _tail_body,
        grid=(_NT // 128,),
        in_specs=[pl.BlockSpec((_D, 128), lambda i: (0, i + base))],
        out_specs=pl.BlockSpec((128, 2 * _D), lambda i: (i, 0)),
        out_shape=jax.ShapeDtypeStruct((_NT, 2 * _D), jnp.float32),
    )(t)


def _sqrt16(x):
    y = jnp.full((_LANES,), 0.7, jnp.float32)
    for _ in range(8):
        y = 0.5 * (y + x / y)
    return y


def _row_sums(vregs):
    iota = lax.iota(jnp.int32, _LANES)
    half = _LANES // 2
    while len(vregs) > 1:
        mask = (iota & half) == 0
        perm = iota ^ half
        nxt = []
        for m in range(0, len(vregs), 2):
            a, b = vregs[m], vregs[m + 1]
            u = jnp.where(mask, a, b)
            w = jnp.where(mask, b, a)
            nxt.append(u + jnp.take(w, perm))
        vregs = nxt
        half //= 2
    return vregs[0]


def _transe_body(head_hbm, rel_hbm, tail_hbm, ent_hbm, reltab_hbm, tailtab_hbm,
                 out_hbm, ih, ir, it, ihp, irp, itp, hv, rv, tv, rowbuf, ov,
                 sems):
    wid = lax.axis_index("s") * _NC + lax.axis_index("c")
    base = wid * _BPW

    pltpu.sync_copy(head_hbm.at[pl.ds(base, _BPW)], ih)
    pltpu.sync_copy(rel_hbm.at[pl.ds(base, _BPW)], ir)
    pltpu.sync_copy(tail_hbm.at[pl.ds(base, _BPW)], it)

    def fold(i, carry):
        s = pl.ds(i * _LANES, _LANES)
        vh = ih[s]
        vr = ir[s]
        vt = it[s]
        ihp[s] = jnp.where(vh >= _KE, vh - _KE, vh)
        irp[s] = jnp.where(vr >= _KR, vr - _KR, vr)
        itp[s] = jnp.where(vt >= _KE, vt - _KE, vt)
        return carry

    lax.fori_loop(0, _BPW // _LANES, fold, 0)

    def start_chunk(c):
        slot = c % 2
        rows = pl.ds(c * _CHUNK, _CHUNK)
        return [
            pltpu.async_copy(ent_hbm.at[ihp.at[rows]], hv.at[slot], sems.at[slot, 0]),
            pltpu.async_copy(reltab_hbm.at[irp.at[rows]], rv.at[slot], sems.at[slot, 1]),
            pltpu.async_copy(ent_hbm.at[itp.at[rows]], tv.at[slot], sems.at[slot, 2]),
        ]

    inflight = start_chunk(0)
    for c in range(_NCHUNK):
        nxt = start_chunk(c + 1) if c + 1 < _NCHUNK else None
        for cp in inflight:
            cp.wait()
        slot = c % 2
        h2, r2, t2 = hv.at[slot], rv.at[slot], tv.at[slot]

        def group(g, carry):
            off = c * _CHUNK + g * _LANES
            vh = ih[pl.ds(off, _LANES)]
            vr = ir[pl.ds(off, _LANES)]
            vt = it[pl.ds(off, _LANES)]
            sq = []
            for m in range(_LANES):
                row = g * _LANES + _BITREV[m]
                oh = jnp.where(vh[_BITREV[m]] >= _KE, _D, 0)
                orr = jnp.where(vr[_BITREV[m]] >= _KR, _D, 0)
                ot = jnp.where(vt[_BITREV[m]] >= _KE, _D, 0)
                acc = None
                for k in range(_D // _LANES):
                    hseg = h2[row, pl.ds(oh + k * _LANES, _LANES)]
                    rseg = r2[row, pl.ds(orr + k * _LANES, _LANES)]
                    tseg = t2[row, pl.ds(ot + k * _LANES, _LANES)]
                    df = (hseg - tseg) + rseg
                    sq2 = df * df
                    acc = sq2 if acc is None else acc + sq2
                sq.append(acc)
            ov[pl.ds(c * _CHUNK + g * _LANES, _LANES)] = -_sqrt16(_row_sums(sq))
            return carry

        lax.fori_loop(0, _CHUNK // _LANES, group, 0)
        if nxt is not None:
            inflight = nxt

    iota = lax.iota(jnp.int32, _LANES)

    def fixgrp(g, carry):
        s = pl.ds(g * _LANES, _LANES)
        vh = ih[s]
        vt = it[s]

        anyv = jnp.where(jnp.logical_or(vh >= _E0, vt >= _E0), 1, 0)
        for hh in (8, 4, 2, 1):
            anyv = anyv + jnp.take(anyv, iota ^ hh)

        @pl.when(anyv[0] > 0)
        def _():
            vr = ir[s]
            for l in range(_LANES):
                rh = vh[l]
                rr = vr[l]
                rt = vt[l]

                @pl.when(jnp.logical_or(rh >= _E0, rt >= _E0))
                def _():
                    @pl.when(rh >= _E0)
                    def _():
                        pltpu.sync_copy(tailtab_hbm.at[rh - _E0], rowbuf.at[0])

                    @pl.when(rh < _E0)
                    def _():
                        pltpu.sync_copy(
                            ent_hbm.at[jnp.where(rh >= _KE, rh - _KE, rh)],
                            rowbuf.at[0])

                    pltpu.sync_copy(
                        reltab_hbm.at[jnp.where(rr >= _KR, rr - _KR, rr)],
                        rowbuf.at[1])

                    @pl.when(rt >= _E0)
                    def _():
                        pltpu.sync_copy(tailtab_hbm.at[rt - _E0], rowbuf.at[2])

                    @pl.when(rt < _E0)
                    def _():
                        pltpu.sync_copy(
                            ent_hbm.at[jnp.where(rt >= _KE, rt - _KE, rt)],
                            rowbuf.at[2])

                    oh = jnp.where(jnp.logical_or(rh >= _E0, rh < _KE), 0, _D)
                    orr = jnp.where(rr >= _KR, _D, 0)
                    ot = jnp.where(jnp.logical_or(rt >= _E0, rt < _KE), 0, _D)
                    acc = jnp.zeros((_LANES,), jnp.float32)
                    for k in range(_D // _LANES):
                        h = rowbuf[0, pl.ds(oh + k * _LANES, _LANES)]
                        r = rowbuf[1, pl.ds(orr + k * _LANES, _LANES)]
                        t = rowbuf[2, pl.ds(ot + k * _LANES, _LANES)]
                        df = (h - t) + r
                        acc = acc + df * df
                    for hh in (8, 4, 2, 1):
                        acc = acc + jnp.take(acc, iota ^ hh)
                    val = -_sqrt16(acc)
                    seg = ov[s]
                    ov[s] = jnp.where(iota == l, val, seg)
        return carry

    lax.fori_loop(0, _BPW // _LANES, fixgrp, 0)

    pltpu.sync_copy(ov, out_hbm.at[pl.ds(base, _BPW)])


@jax.jit
def kernel(head, relation, tail, entity_table, relation_table):
    ent2 = _two_halves(entity_table, _KE, _BE, 244)
    rel2 = _two_halves(relation_table, _KR, _KR, 2)
    ent_tail = _tail_table(entity_table)
    mesh = plsc.VectorSubcoreMesh(core_axis_name="c", subcore_axis_name="s")
    f = functools.partial(
        pl.kernel,
        out_type=jax.ShapeDtypeStruct((_BATCH,), jnp.float32),
        mesh=mesh,
        scratch_types=[
            pltpu.VMEM((_BPW,), jnp.int32),
            pltpu.VMEM((_BPW,), jnp.int32),
            pltpu.VMEM((_BPW,), jnp.int32),
            pltpu.VMEM((_BPW,), jnp.int32),
            pltpu.VMEM((_BPW,), jnp.int32),
            pltpu.VMEM((_BPW,), jnp.int32),
            pltpu.VMEM((2, _CHUNK, 2 * _D), jnp.float32),
            pltpu.VMEM((2, _CHUNK, 2 * _D), jnp.float32),
            pltpu.VMEM((2, _CHUNK, 2 * _D), jnp.float32),
            pltpu.VMEM((3, 2 * _D), jnp.float32),
            pltpu.VMEM((_BPW,), jnp.float32),
            pltpu.SemaphoreType.DMA((2, 3)),
        ],
    )(_transe_body)
    return f(head, relation, tail, ent2, rel2, ent_tail)


# speedup vs baseline: 2.0265x; 2.0265x over previous
import functools

import jax
import jax.numpy as jnp
from jax import lax
from jax.experimental import pallas as pl
from jax.experimental.pallas import tpu as pltpu
from jax.experimental.pallas import tpu_sc as plsc

_BATCH = 16384
_D = 64
_LANES = 16
_NC = 2
_NS = 16
_NW = _NC * _NS
_BPW = _BATCH // _NW
_CHUNK = 128
_NCHUNK = _BPW // _CHUNK

_NE = 1000000
_BE = 8192
_KE = 507904
_E0 = 999424
_NT = 640
_KR = 512

_BITREV = [0, 8, 4, 12, 2, 10, 6, 14, 1, 9, 5, 13, 3, 11, 7, 15]


def _eye64():
    return (lax.broadcasted_iota(jnp.int32, (_D, _D), 0)
            == lax.broadcasted_iota(jnp.int32, (_D, _D), 1)).astype(jnp.float32)


def _mxu_t(x):
    return lax.dot_general(x, _eye64(), (((0,), (0,)), ((), ())),
                           precision=lax.Precision.HIGHEST,
                           preferred_element_type=jnp.float32)


def _relayout_body(a_ref, b_ref, o_ref):
    o_ref[...] = jnp.concatenate([a_ref[...].T, b_ref[...].T], axis=1)


def _two_halves(table, k, be, nblk_in):
    t = table.T
    grid = k // be
    return pl.pallas_call(
        _relayout_body,
        grid=(grid,),
        in_specs=[pl.BlockSpec((_D, be), lambda i: (0, i)),
                  pl.BlockSpec((_D, be),
                               lambda i: (0, jnp.minimum(i + grid, nblk_in - 1)))],
        out_specs=pl.BlockSpec((be, 2 * _D), lambda i: (i, 0)),
        out_shape=jax.ShapeDtypeStruct((k, 2 * _D), jnp.float32),
    )(t, t)


def _tail_body(a_ref, o_ref):
    o_ref[...] = jnp.concatenate(
        [a_ref[...].T, jnp.zeros((128, _D), jnp.float32)], axis=1)


def _tail_table(table):
    t = table.T
    base = _E0 // 128
    return pl.pallas_call(
        _tail_body,
        grid=(_NT // 128,),
        in_specs=[pl.BlockSpec((_D, 128), lambda i: (0, i + base))],
        out_specs=pl.BlockSpec((128, 2 * _D), lambda i: (i, 0)),
        out_shape=jax.ShapeDtypeStruct((_NT, 2 * _D), jnp.float32),
    )(t)


def _sqrt16(x):
    y = jnp.full((_LANES,), 0.7, jnp.float32)
    for _ in range(8):
        y = 0.5 * (y + x / y)
    return y


def _row_sums(vregs):
    iota = lax.iota(jnp.int32, _LANES)
    half = _LANES // 2
    while len(vregs) > 1:
        mask = (iota & half) == 0
        perm = iota ^ half
        nxt = []
        for m in range(0, len(vregs), 2):
            a, b = vregs[m], vregs[m + 1]
            u = jnp.where(mask, a, b)
            w = jnp.where(mask, b, a)
            nxt.append(u + jnp.take(w, perm))
        vregs = nxt
        half //= 2
    return vregs[0]


def _transe_body(head_hbm, rel_hbm, tail_hbm, ent_hbm, reltab_hbm, tailtab_hbm,
                 out_hbm, ih, ir, it, ihp, irp, itp, hv, rv, tv, rowbuf, ov,
                 sems):
    wid = lax.axis_index("s") * _NC + lax.axis_index("c")
    base = wid * _BPW

    pltpu.sync_copy(head_hbm.at[pl.ds(base, _BPW)], ih)
    pltpu.sync_copy(rel_hbm.at[pl.ds(base, _BPW)], ir)
    pltpu.sync_copy(tail_hbm.at[pl.ds(base, _BPW)], it)

    def fold(i, carry):
        s = pl.ds(i * _LANES, _LANES)
        vh = ih[s]
        vr = ir[s]
        vt = it[s]
        ihp[s] = jnp.where(vh >= _KE, vh - _KE, vh)
        irp[s] = jnp.where(vr >= _KR, vr - _KR, vr)
        itp[s] = jnp.where(vt >= _KE, vt - _KE, vt)
        return carry

    lax.fori_loop(0, _BPW // _LANES, fold, 0)

    def start_chunk(c):
        slot = c % 2
        rows = pl.ds(c * _CHUNK, _CHUNK)
        return [
            pltpu.async_copy(ent_hbm.at[ihp.at[rows]], hv.at[slot], sems.at[slot, 0]),
            pltpu.async_copy(reltab_hbm.at[irp.at[rows]], rv.at[slot], sems.at[slot, 1]),
            pltpu.async_copy(ent_hbm.at[itp.at[rows]], tv.at[slot], sems.at[slot, 2]),
        ]

    inflight = start_chunk(0)
    for c in range(_NCHUNK):
        nxt = start_chunk(c + 1) if c + 1 < _NCHUNK else None
        for cp in inflight:
            cp.wait()
        slot = c % 2
        h2, r2, t2 = hv.at[slot], rv.at[slot], tv.at[slot]

        def group(g, carry):
            off = c * _CHUNK + g * _LANES
            vh = ih[pl.ds(off, _LANES)]
            vr = ir[pl.ds(off, _LANES)]
            vt = it[pl.ds(off, _LANES)]
            sq = []
            for m in range(_LANES):
                row = g * _LANES + _BITREV[m]
                oh = jnp.where(vh[_BITREV[m]] >= _KE, _D, 0)
                orr = jnp.where(vr[_BITREV[m]] >= _KR, _D, 0)
                ot = jnp.where(vt[_BITREV[m]] >= _KE, _D, 0)
                acc = None
                for k in range(_D // _LANES):
                    hseg = h2[row, pl.ds(oh + k * _LANES, _LANES)]
                    rseg = r2[row, pl.ds(orr + k * _LANES, _LANES)]
                    tseg = t2[row, pl.ds(ot + k * _LANES, _LANES)]
                    df = (hseg - tseg) + rseg
                    sq2 = df * df
                    acc = sq2 if acc is None else acc + sq2
                sq.append(acc)
            ov[pl.ds(c * _CHUNK + g * _LANES, _LANES)] = -_sqrt16(_row_sums(sq))
            return carry

        lax.fori_loop(0, _CHUNK // _LANES, group, 0)
        if nxt is not None:
            inflight = nxt

    iota = lax.iota(jnp.int32, _LANES)

    def fixgrp(g, carry):
        s = pl.ds(g * _LANES, _LANES)
        vh = ih[s]
        vt = it[s]

        anyv = jnp.where(jnp.logical_or(vh >= _E0, vt >= _E0), 1, 0)
        for hh in (8, 4, 2, 1):
            anyv = anyv + jnp.take(anyv, iota ^ hh)

        @pl.when(anyv[0] > 0)
        def _():
            vr = ir[s]
            for l in range(_LANES):
                rh = vh[l]
                rr = vr[l]
                rt = vt[l]

                @pl.when(jnp.logical_or(rh >= _E0, rt >= _E0))
                def _():
                    @pl.when(rh >= _E0)
                    def _():
                        pltpu.sync_copy(tailtab_hbm.at[rh - _E0], rowbuf.at[0])

                    @pl.when(rh < _E0)
                    def _():
                        pltpu.sync_copy(
                            ent_hbm.at[jnp.where(rh >= _KE, rh - _KE, rh)],
                            rowbuf.at[0])

                    pltpu.sync_copy(
                        reltab_hbm.at[jnp.where(rr >= _KR, rr - _KR, rr)],
                        rowbuf.at[1])

                    @pl.when(rt >= _E0)
                    def _():
                        pltpu.sync_copy(tailtab_hbm.at[rt - _E0], rowbuf.at[2])

                    @pl.when(rt < _E0)
                    def _():
                        pltpu.sync_copy(
                            ent_hbm.at[jnp.where(rt >= _KE, rt - _KE, rt)],
                            rowbuf.at[2])

                    oh = jnp.where(jnp.logical_or(rh >= _E0, rh < _KE), 0, _D)
                    orr = jnp.where(rr >= _KR, _D, 0)
                    ot = jnp.where(jnp.logical_or(rt >= _E0, rt < _KE), 0, _D)
                    acc = jnp.zeros((_LANES,), jnp.float32)
                    for k in range(_D // _LANES):
                        h = rowbuf[0, pl.ds(oh + k * _LANES, _LANES)]
                        r = rowbuf[1, pl.ds(orr + k * _LANES, _LANES)]
                        t = rowbuf[2, pl.ds(ot + k * _LANES, _LANES)]
                        df = (h - t) + r
                        acc = acc + df * df
                    for hh in (8, 4, 2, 1):
                        acc = acc + jnp.take(acc, iota ^ hh)
                    val = -_sqrt16(acc)
                    seg = ov[s]
                    ov[s] = jnp.where(iota == l, val, seg)
        return carry

    lax.fori_loop(0, _BPW // _LANES, fixgrp, 0)

    pltpu.sync_copy(ov, out_hbm.at[pl.ds(base, _BPW)])


@jax.jit
def kernel(head, relation, tail, entity_table, relation_table):
    ent2 = _two_halves(entity_table, _KE, _BE, 122)
    rel2 = _two_halves(relation_table, _KR, _KR, 2)
    ent_tail = _tail_table(entity_table)
    mesh = plsc.VectorSubcoreMesh(core_axis_name="c", subcore_axis_name="s")
    f = functools.partial(
        pl.kernel,
        out_type=jax.ShapeDtypeStruct((_BATCH,), jnp.float32),
        mesh=mesh,
        scratch_types=[
            pltpu.VMEM((_BPW,), jnp.int32),
            pltpu.VMEM((_BPW,), jnp.int32),
            pltpu.VMEM((_BPW,), jnp.int32),
            pltpu.VMEM((_BPW,), jnp.int32),
            pltpu.VMEM((_BPW,), jnp.int32),
            pltpu.VMEM((_BPW,), jnp.int32),
            pltpu.VMEM((2, _CHUNK, 2 * _D), jnp.float32),
            pltpu.VMEM((2, _CHUNK, 2 * _D), jnp.float32),
            pltpu.VMEM((2, _CHUNK, 2 * _D), jnp.float32),
            pltpu.VMEM((3, 2 * _D), jnp.float32),
            pltpu.VMEM((_BPW,), jnp.float32),
            pltpu.SemaphoreType.DMA((2, 3)),
        ],
    )(_transe_body)
    return f(head, relation, tail, ent2, rel2, ent_tail)


# speedup vs baseline: 2.1361x; 1.0541x over previous
import functools

import jax
import jax.numpy as jnp
from jax import lax
from jax.experimental import pallas as pl
from jax.experimental.pallas import tpu as pltpu
from jax.experimental.pallas import tpu_sc as plsc

_BATCH = 16384
_D = 64
_LANES = 16
_NC = 2
_NS = 16
_NW = _NC * _NS
_BPW = _BATCH // _NW
_CHUNK = 128
_NCHUNK = _BPW // _CHUNK

_NE = 1000000
_BE = 16384
_KE = 507904
_E0 = 999424
_NT = 640
_KR = 512

_BITREV = [0, 8, 4, 12, 2, 10, 6, 14, 1, 9, 5, 13, 3, 11, 7, 15]


def _eye64():
    return (lax.broadcasted_iota(jnp.int32, (_D, _D), 0)
            == lax.broadcasted_iota(jnp.int32, (_D, _D), 1)).astype(jnp.float32)


def _mxu_t(x):
    return lax.dot_general(x, _eye64(), (((0,), (0,)), ((), ())),
                           precision=lax.Precision.HIGHEST,
                           preferred_element_type=jnp.float32)


def _relayout_body(a_ref, b_ref, o_ref):
    o_ref[...] = jnp.concatenate([a_ref[...].T, b_ref[...].T], axis=1)


def _two_halves(table, k, be, nblk_in):
    t = table.T
    grid = k // be
    return pl.pallas_call(
        _relayout_body,
        grid=(grid,),
        in_specs=[pl.BlockSpec((_D, be), lambda i: (0, i)),
                  pl.BlockSpec((_D, be),
                               lambda i: (0, jnp.minimum(i + grid, nblk_in - 1)))],
        out_specs=pl.BlockSpec((be, 2 * _D), lambda i: (i, 0)),
        out_shape=jax.ShapeDtypeStruct((k, 2 * _D), jnp.float32),
    )(t, t)


def _tail_body(a_ref, o_ref):
    o_ref[...] = jnp.concatenate(
        [a_ref[...].T, jnp.zeros((128, _D), jnp.float32)], axis=1)


def _tail_table(table):
    t = table.T
    base = _E0 // 128
    return pl.pallas_call(
        _tail_body,
        grid=(_NT // 128,),
        in_specs=[pl.BlockSpec((_D, 128), lambda i: (0, i + base))],
        out_specs=pl.BlockSpec((128, 2 * _D), lambda i: (i, 0)),
        out_shape=jax.ShapeDtypeStruct((_NT, 2 * _D), jnp.float32),
    )(t)


def _sqrt16(x):
    y = jnp.full((_LANES,), 0.7, jnp.float32)
    for _ in range(8):
        y = 0.5 * (y + x / y)
    return y


def _row_sums(vregs):
    iota = lax.iota(jnp.int32, _LANES)
    half = _LANES // 2
    while len(vregs) > 1:
        mask = (iota & half) == 0
        perm = iota ^ half
        nxt = []
        for m in range(0, len(vregs), 2):
            a, b = vregs[m], vregs[m + 1]
            u = jnp.where(mask, a, b)
            w = jnp.where(mask, b, a)
            nxt.append(u + jnp.take(w, perm))
        vregs = nxt
        half //= 2
    return vregs[0]


def _transe_body(head_hbm, rel_hbm, tail_hbm, ent_hbm, reltab_hbm, tailtab_hbm,
                 out_hbm, ih, ir, it, ihp, irp, itp, hv, rv, tv, rowbuf, ov,
                 sems):
    wid = lax.axis_index("s") * _NC + lax.axis_index("c")
    base = wid * _BPW

    pltpu.sync_copy(head_hbm.at[pl.ds(base, _BPW)], ih)
    pltpu.sync_copy(rel_hbm.at[pl.ds(base, _BPW)], ir)
    pltpu.sync_copy(tail_hbm.at[pl.ds(base, _BPW)], it)

    def fold(i, carry):
        s = pl.ds(i * _LANES, _LANES)
        vh = ih[s]
        vr = ir[s]
        vt = it[s]
        ihp[s] = jnp.where(vh >= _KE, vh - _KE, vh)
        irp[s] = jnp.where(vr >= _KR, vr - _KR, vr)
        itp[s] = jnp.where(vt >= _KE, vt - _KE, vt)
        return carry

    lax.fori_loop(0, _BPW // _LANES, fold, 0)

    def start_chunk(c):
        slot = c % 2
        rows = pl.ds(c * _CHUNK, _CHUNK)
        return [
            pltpu.async_copy(ent_hbm.at[ihp.at[rows]], hv.at[slot], sems.at[slot, 0]),
            pltpu.async_copy(reltab_hbm.at[irp.at[rows]], rv.at[slot], sems.at[slot, 1]),
            pltpu.async_copy(ent_hbm.at[itp.at[rows]], tv.at[slot], sems.at[slot, 2]),
        ]

    inflight = start_chunk(0)
    for c in range(_NCHUNK):
        nxt = start_chunk(c + 1) if c + 1 < _NCHUNK else None
        for cp in inflight:
            cp.wait()
        slot = c % 2
        h2, r2, t2 = hv.at[slot], rv.at[slot], tv.at[slot]

        def group(g, carry):
            off = c * _CHUNK + g * _LANES
            vh = ih[pl.ds(off, _LANES)]
            vr = ir[pl.ds(off, _LANES)]
            vt = it[pl.ds(off, _LANES)]
            sq = []
            for m in range(_LANES):
                row = g * _LANES + _BITREV[m]
                oh = jnp.where(vh[_BITREV[m]] >= _KE, _D, 0)
                orr = jnp.where(vr[_BITREV[m]] >= _KR, _D, 0)
                ot = jnp.where(vt[_BITREV[m]] >= _KE, _D, 0)
                acc = None
                for k in range(_D // _LANES):
                    hseg = h2[row, pl.ds(oh + k * _LANES, _LANES)]
                    rseg = r2[row, pl.ds(orr + k * _LANES, _LANES)]
                    tseg = t2[row, pl.ds(ot + k * _LANES, _LANES)]
                    df = (hseg - tseg) + rseg
                    sq2 = df * df
                    acc = sq2 if acc is None else acc + sq2
                sq.append(acc)
            ov[pl.ds(c * _CHUNK + g * _LANES, _LANES)] = -_sqrt16(_row_sums(sq))
            return carry

        lax.fori_loop(0, _CHUNK // _LANES, group, 0)
        if nxt is not None:
            inflight = nxt

    iota = lax.iota(jnp.int32, _LANES)

    def fixgrp(g, carry):
        s = pl.ds(g * _LANES, _LANES)
        vh = ih[s]
        vt = it[s]

        anyv = jnp.where(jnp.logical_or(vh >= _E0, vt >= _E0), 1, 0)
        for hh in (8, 4, 2, 1):
            anyv = anyv + jnp.take(anyv, iota ^ hh)

        @pl.when(anyv[0] > 0)
        def _():
            vr = ir[s]
            for l in range(_LANES):
                rh = vh[l]
                rr = vr[l]
                rt = vt[l]

                @pl.when(jnp.logical_or(rh >= _E0, rt >= _E0))
                def _():
                    @pl.when(rh >= _E0)
                    def _():
                        pltpu.sync_copy(tailtab_hbm.at[rh - _E0], rowbuf.at[0])

                    @pl.when(rh < _E0)
                    def _():
                        pltpu.sync_copy(
                            ent_hbm.at[jnp.where(rh >= _KE, rh - _KE, rh)],
                            rowbuf.at[0])

                    pltpu.sync_copy(
                        reltab_hbm.at[jnp.where(rr >= _KR, rr - _KR, rr)],
                        rowbuf.at[1])

                    @pl.when(rt >= _E0)
                    def _():
                        pltpu.sync_copy(tailtab_hbm.at[rt - _E0], rowbuf.at[2])

                    @pl.when(rt < _E0)
                    def _():
                        pltpu.sync_copy(
                            ent_hbm.at[jnp.where(rt >= _KE, rt - _KE, rt)],
                            rowbuf.at[2])

                    oh = jnp.where(jnp.logical_or(rh >= _E0, rh < _KE), 0, _D)
                    orr = jnp.where(rr >= _KR, _D, 0)
                    ot = jnp.where(jnp.logical_or(rt >= _E0, rt < _KE), 0, _D)
                    acc = jnp.zeros((_LANES,), jnp.float32)
                    for k in range(_D // _LANES):
                        h = rowbuf[0, pl.ds(oh + k * _LANES, _LANES)]
                        r = rowbuf[1, pl.ds(orr + k * _LANES, _LANES)]
                        t = rowbuf[2, pl.ds(ot + k * _LANES, _LANES)]
                        df = (h - t) + r
                        acc = acc + df * df
                    for hh in (8, 4, 2, 1):
                        acc = acc + jnp.take(acc, iota ^ hh)
                    val = -_sqrt16(acc)
                    seg = ov[s]
                    ov[s] = jnp.where(iota == l, val, seg)
        return carry

    lax.fori_loop(0, _BPW // _LANES, fixgrp, 0)

    pltpu.sync_copy(ov, out_hbm.at[pl.ds(base, _BPW)])


@jax.jit
def kernel(head, relation, tail, entity_table, relation_table):
    ent2 = _two_halves(entity_table, _KE, _BE, 61)
    rel2 = _two_halves(relation_table, _KR, _KR, 2)
    ent_tail = _tail_table(entity_table)
    mesh = plsc.VectorSubcoreMesh(core_axis_name="c", subcore_axis_name="s")
    f = functools.partial(
        pl.kernel,
        out_type=jax.ShapeDtypeStruct((_BATCH,), jnp.float32),
        mesh=mesh,
        scratch_types=[
            pltpu.VMEM((_BPW,), jnp.int32),
            pltpu.VMEM((_BPW,), jnp.int32),
            pltpu.VMEM((_BPW,), jnp.int32),
            pltpu.VMEM((_BPW,), jnp.int32),
            pltpu.VMEM((_BPW,), jnp.int32),
            pltpu.VMEM((_BPW,), jnp.int32),
            pltpu.VMEM((2, _CHUNK, 2 * _D), jnp.float32),
            pltpu.VMEM((2, _CHUNK, 2 * _D), jnp.float32),
            pltpu.VMEM((2, _CHUNK, 2 * _D), jnp.float32),
            pltpu.VMEM((3, 2 * _D), jnp.float32),
            pltpu.VMEM((_BPW,), jnp.float32),
            pltpu.SemaphoreType.DMA((2, 3)),
        ],
    )(_transe_body)
    return f(head, relation, tail, ent2, rel2, ent_tail)


# speedup vs baseline: 2.1605x; 1.0114x over previous
import functools

import jax
import jax.numpy as jnp
from jax import lax
from jax.experimental import pallas as pl
from jax.experimental.pallas import tpu as pltpu
from jax.experimental.pallas import tpu_sc as plsc

_BATCH = 16384
_D = 64
_LANES = 16
_NC = 2
_NS = 16
_NW = _NC * _NS
_BPW = _BATCH // _NW
_CHUNK = 128
_NCHUNK = _BPW // _CHUNK

_NE = 1000000
_BE = 16384
_KE = 507904
_E0 = 999424
_NT = 640
_KR = 512

_BITREV = [0, 8, 4, 12, 2, 10, 6, 14, 1, 9, 5, 13, 3, 11, 7, 15]


_GRID = _KE // _BE
_NBLK_IN = 61
_TBASE = _E0 // 128


def _relayout_body(a_ref, b_ref, ra_ref, rb_ref,
                   t0_ref, t1_ref, t2_ref, t3_ref, t4_ref,
                   oe_ref, or_ref, ot_ref):
    oe_ref[...] = jnp.concatenate([a_ref[...].T, b_ref[...].T], axis=1)

    @pl.when(pl.program_id(0) == 0)
    def _():
        or_ref[...] = jnp.concatenate([ra_ref[...].T, rb_ref[...].T], axis=1)
        z = jnp.zeros((128, _D), jnp.float32)
        ot_ref[...] = jnp.concatenate(
            [jnp.concatenate([t.T, z], axis=1)
             for t in (t0_ref[...], t1_ref[...], t2_ref[...], t3_ref[...],
                       t4_ref[...])], axis=0)


def _relayout_all(ent, rel):
    t = ent.T
    tr = rel.T
    return pl.pallas_call(
        _relayout_body,
        grid=(_GRID,),
        in_specs=[
            pl.BlockSpec((_D, _BE), lambda i: (0, i)),
            pl.BlockSpec((_D, _BE),
                         lambda i: (0, jnp.minimum(i + _GRID, _NBLK_IN - 1))),
            pl.BlockSpec((_D, _KR), lambda i: (0, 0)),
            pl.BlockSpec((_D, _KR), lambda i: (0, 1)),
            pl.BlockSpec((_D, 128), lambda i: (0, _TBASE)),
            pl.BlockSpec((_D, 128), lambda i: (0, _TBASE + 1)),
            pl.BlockSpec((_D, 128), lambda i: (0, _TBASE + 2)),
            pl.BlockSpec((_D, 128), lambda i: (0, _TBASE + 3)),
            pl.BlockSpec((_D, 128), lambda i: (0, _TBASE + 4)),
        ],
        out_specs=[
            pl.BlockSpec((_BE, 2 * _D), lambda i: (i, 0)),
            pl.BlockSpec((_KR, 2 * _D), lambda i: (0, 0)),
            pl.BlockSpec((_NT, 2 * _D), lambda i: (0, 0)),
        ],
        out_shape=[
            jax.ShapeDtypeStruct((_KE, 2 * _D), jnp.float32),
            jax.ShapeDtypeStruct((_KR, 2 * _D), jnp.float32),
            jax.ShapeDtypeStruct((_NT, 2 * _D), jnp.float32),
        ],
    )(t, t, tr, tr, t, t, t, t, t)


def _sqrt16(x):
    y = jnp.full((_LANES,), 0.7, jnp.float32)
    for _ in range(8):
        y = 0.5 * (y + x / y)
    return y


def _row_sums(vregs):
    iota = lax.iota(jnp.int32, _LANES)
    half = _LANES // 2
    while len(vregs) > 1:
        mask = (iota & half) == 0
        perm = iota ^ half
        nxt = []
        for m in range(0, len(vregs), 2):
            a, b = vregs[m], vregs[m + 1]
            u = jnp.where(mask, a, b)
            w = jnp.where(mask, b, a)
            nxt.append(u + jnp.take(w, perm))
        vregs = nxt
        half //= 2
    return vregs[0]


def _transe_body(head_hbm, rel_hbm, tail_hbm, ent_hbm, reltab_hbm, tailtab_hbm,
                 out_hbm, ih, ir, it, ihp, irp, itp, hv, rv, tv, rowbuf, ov,
                 sems):
    wid = lax.axis_index("s") * _NC + lax.axis_index("c")
    base = wid * _BPW

    pltpu.sync_copy(head_hbm.at[pl.ds(base, _BPW)], ih)
    pltpu.sync_copy(rel_hbm.at[pl.ds(base, _BPW)], ir)
    pltpu.sync_copy(tail_hbm.at[pl.ds(base, _BPW)], it)

    def fold(i, carry):
        s = pl.ds(i * _LANES, _LANES)
        vh = ih[s]
        vr = ir[s]
        vt = it[s]
        ihp[s] = jnp.where(vh >= _KE, vh - _KE, vh)
        irp[s] = jnp.where(vr >= _KR, vr - _KR, vr)
        itp[s] = jnp.where(vt >= _KE, vt - _KE, vt)
        return carry

    lax.fori_loop(0, _BPW // _LANES, fold, 0)

    def start_chunk(c):
        slot = c % 2
        rows = pl.ds(c * _CHUNK, _CHUNK)
        return [
            pltpu.async_copy(ent_hbm.at[ihp.at[rows]], hv.at[slot], sems.at[slot, 0]),
            pltpu.async_copy(reltab_hbm.at[irp.at[rows]], rv.at[slot], sems.at[slot, 1]),
            pltpu.async_copy(ent_hbm.at[itp.at[rows]], tv.at[slot], sems.at[slot, 2]),
        ]

    inflight = start_chunk(0)
    for c in range(_NCHUNK):
        nxt = start_chunk(c + 1) if c + 1 < _NCHUNK else None
        for cp in inflight:
            cp.wait()
        slot = c % 2
        h2, r2, t2 = hv.at[slot], rv.at[slot], tv.at[slot]

        def group(g, carry):
            off = c * _CHUNK + g * _LANES
            vh = ih[pl.ds(off, _LANES)]
            vr = ir[pl.ds(off, _LANES)]
            vt = it[pl.ds(off, _LANES)]
            sq = []
            for m in range(_LANES):
                row = g * _LANES + _BITREV[m]
                oh = jnp.where(vh[_BITREV[m]] >= _KE, _D, 0)
                orr = jnp.where(vr[_BITREV[m]] >= _KR, _D, 0)
                ot = jnp.where(vt[_BITREV[m]] >= _KE, _D, 0)
                acc = None
                for k in range(_D // _LANES):
                    hseg = h2[row, pl.ds(oh + k * _LANES, _LANES)]
                    rseg = r2[row, pl.ds(orr + k * _LANES, _LANES)]
                    tseg = t2[row, pl.ds(ot + k * _LANES, _LANES)]
                    df = (hseg - tseg) + rseg
                    sq2 = df * df
                    acc = sq2 if acc is None else acc + sq2
                sq.append(acc)
            ov[pl.ds(c * _CHUNK + g * _LANES, _LANES)] = -_sqrt16(_row_sums(sq))
            return carry

        lax.fori_loop(0, _CHUNK // _LANES, group, 0)
        if nxt is not None:
            inflight = nxt

    iota = lax.iota(jnp.int32, _LANES)

    def fixgrp(g, carry):
        s = pl.ds(g * _LANES, _LANES)
        vh = ih[s]
        vt = it[s]

        anyv = jnp.where(jnp.logical_or(vh >= _E0, vt >= _E0), 1, 0)
        for hh in (8, 4, 2, 1):
            anyv = anyv + jnp.take(anyv, iota ^ hh)

        @pl.when(anyv[0] > 0)
        def _():
            vr = ir[s]
            for l in range(_LANES):
                rh = vh[l]
                rr = vr[l]
                rt = vt[l]

                @pl.when(jnp.logical_or(rh >= _E0, rt >= _E0))
                def _():
                    @pl.when(rh >= _E0)
                    def _():
                        pltpu.sync_copy(tailtab_hbm.at[rh - _E0], rowbuf.at[0])

                    @pl.when(rh < _E0)
                    def _():
                        pltpu.sync_copy(
                            ent_hbm.at[jnp.where(rh >= _KE, rh - _KE, rh)],
                            rowbuf.at[0])

                    pltpu.sync_copy(
                        reltab_hbm.at[jnp.where(rr >= _KR, rr - _KR, rr)],
                        rowbuf.at[1])

                    @pl.when(rt >= _E0)
                    def _():
                        pltpu.sync_copy(tailtab_hbm.at[rt - _E0], rowbuf.at[2])

                    @pl.when(rt < _E0)
                    def _():
                        pltpu.sync_copy(
                            ent_hbm.at[jnp.where(rt >= _KE, rt - _KE, rt)],
                            rowbuf.at[2])

                    oh = jnp.where(jnp.logical_or(rh >= _E0, rh < _KE), 0, _D)
                    orr = jnp.where(rr >= _KR, _D, 0)
                    ot = jnp.where(jnp.logical_or(rt >= _E0, rt < _KE), 0, _D)
                    acc = jnp.zeros((_LANES,), jnp.float32)
                    for k in range(_D // _LANES):
                        h = rowbuf[0, pl.ds(oh + k * _LANES, _LANES)]
                        r = rowbuf[1, pl.ds(orr + k * _LANES, _LANES)]
                        t = rowbuf[2, pl.ds(ot + k * _LANES, _LANES)]
                        df = (h - t) + r
                        acc = acc + df * df
                    for hh in (8, 4, 2, 1):
                        acc = acc + jnp.take(acc, iota ^ hh)
                    val = -_sqrt16(acc)
                    seg = ov[s]
                    ov[s] = jnp.where(iota == l, val, seg)
        return carry

    lax.fori_loop(0, _BPW // _LANES, fixgrp, 0)

    pltpu.sync_copy(ov, out_hbm.at[pl.ds(base, _BPW)])


@jax.jit
def kernel(head, relation, tail, entity_table, relation_table):
    ent2, rel2, ent_tail = _relayout_all(entity_table, relation_table)
    mesh = plsc.VectorSubcoreMesh(core_axis_name="c", subcore_axis_name="s")
    f = functools.partial(
        pl.kernel,
        out_type=jax.ShapeDtypeStruct((_BATCH,), jnp.float32),
        mesh=mesh,
        scratch_types=[
            pltpu.VMEM((_BPW,), jnp.int32),
            pltpu.VMEM((_BPW,), jnp.int32),
            pltpu.VMEM((_BPW,), jnp.int32),
            pltpu.VMEM((_BPW,), jnp.int32),
            pltpu.VMEM((_BPW,), jnp.int32),
            pltpu.VMEM((_BPW,), jnp.int32),
            pltpu.VMEM((2, _CHUNK, 2 * _D), jnp.float32),
            pltpu.VMEM((2, _CHUNK, 2 * _D), jnp.float32),
            pltpu.VMEM((2, _CHUNK, 2 * _D), jnp.float32),
            pltpu.VMEM((3, 2 * _D), jnp.float32),
            pltpu.VMEM((_BPW,), jnp.float32),
            pltpu.SemaphoreType.DMA((2, 3)),
        ],
    )(_transe_body)
    return f(head, relation, tail, ent2, rel2, ent_tail)
